# baseline (speedup 1.0000x reference)
import functools

import jax
import jax.numpy as jnp
from jax import lax
from jax.experimental import pallas as pl
from jax.experimental.pallas import tpu as pltpu
from jax.experimental.pallas import tpu_sc as plsc

_N = 10000
_E = 320000
_D = 128
_H = 128
_OUT = 32
_G = 64

_NCORE = 2
_NSUB = 16
_NW = _NCORE * _NSUB
_NPAD = 10240
_RPS = _NPAD // _NSUB
_CHUNK = 128
_NCH = _E // _CHUNK
_CPW = 80
_HCP = 40
_LASTW = _NCH // _CPW
_LASTN = _NCH - _LASTW * _CPW

_mesh = plsc.VectorSubcoreMesh(core_axis_name="c", subcore_axis_name="s")


@functools.partial(
    pl.kernel,
    out_type=jax.ShapeDtypeStruct((_NCORE, _NPAD), jnp.float32),
    mesh=_mesh,
    scratch_types=[
        pltpu.VMEM((_CPW, _CHUNK), jnp.int32),
        pltpu.VMEM((_CHUNK,), jnp.float32),
        pltpu.VMEM_SHARED((_NPAD,), jnp.float32),
    ],
)
def _deg_pass(edge3d_hbm, zero1_hbm, out_hbm, cidx, ones_v, acc):
    c = lax.axis_index("c")
    s = lax.axis_index("s")
    w = s * _NCORE + c
    for i in range(_CHUNK // 16):
        ones_v[pl.ds(i * 16, 16)] = jnp.ones((16,), jnp.float32)
    pltpu.sync_copy(zero1_hbm, acc.at[pl.ds(s * _RPS, _RPS)])

    @pl.when(w < _LASTW)
    def _():
        pltpu.sync_copy(edge3d_hbm.at[1, pl.ds(w * _CPW, _CPW)], cidx)

    @pl.when(w == _LASTW)
    def _():
        pltpu.sync_copy(edge3d_hbm.at[1, pl.ds(_LASTW * _CPW, _LASTN)],
                        cidx.at[pl.ds(0, _LASTN)])

    plsc.subcore_barrier()

    def body(k, carry):
        pltpu.sync_copy(ones_v, acc.at[cidx.at[k]], add=True)
        return carry

    nch = jnp.where(w < _LASTW, _CPW, _LASTN)
    lax.fori_loop(0, nch, body, 0)
    plsc.subcore_barrier()
    pltpu.sync_copy(acc.at[pl.ds(s * _RPS, _RPS)],
                    out_hbm.at[c, pl.ds(s * _RPS, _RPS)])


@functools.partial(
    pl.kernel,
    out_type=jax.ShapeDtypeStruct((_NCORE, _NPAD, _H), jnp.float32),
    mesh=_mesh,
    scratch_types=[
        pltpu.VMEM((_HCP, _CHUNK), jnp.int32),
        pltpu.VMEM((_HCP, _CHUNK), jnp.int32),
        pltpu.VMEM((_CHUNK, _H), jnp.float32),
        pltpu.VMEM((_CHUNK, _H), jnp.float32),
        pltpu.VMEM_SHARED((_NPAD, _H), jnp.float32),
        pltpu.SemaphoreType.DMA,
        pltpu.SemaphoreType.DMA,
        pltpu.SemaphoreType.DMA,
        pltpu.SemaphoreType.DMA,
    ],
)
def _edge_pass(edge3d_hbm, y_hbm, zero2_hbm, out_hbm,
               ridx, cidx, rows_a, rows_b, acc, sem_a, sem_b, ssa, ssb):
    c = lax.axis_index("c")
    s = lax.axis_index("s")
    w = s * _NCORE + c
    pltpu.sync_copy(zero2_hbm, acc.at[pl.ds(s * _RPS, _RPS)])
    plsc.subcore_barrier()

    def pipeline(n):
        pltpu.async_copy(y_hbm.at[ridx.at[0]], rows_a, sem_a)
        pltpu.async_copy(y_hbm.at[ridx.at[1]], rows_b, sem_b)

        def body(k2, carry):
            kk = k2 * 2
            pltpu.make_async_copy(y_hbm.at[ridx.at[kk]], rows_a,
                                  sem_a).wait()
            pltpu.async_copy(rows_a, acc.at[cidx.at[kk]], ssa, add=True)
            pltpu.make_async_copy(y_hbm.at[ridx.at[kk + 1]], rows_b,
                                  sem_b).wait()
            pltpu.async_copy(rows_b, acc.at[cidx.at[kk + 1]], ssb,
                             add=True)
            pltpu.make_async_copy(rows_a, acc.at[cidx.at[kk]], ssa).wait()
            pltpu.async_copy(y_hbm.at[ridx.at[kk + 2]], rows_a, sem_a)
            pltpu.make_async_copy(rows_b, acc.at[cidx.at[kk + 1]],
                                  ssb).wait()
            pltpu.async_copy(y_hbm.at[ridx.at[kk + 3]], rows_b, sem_b)
            return carry

        lax.fori_loop(0, n // 2 - 1, body, 0)
        pltpu.make_async_copy(y_hbm.at[ridx.at[n - 2]], rows_a,
                              sem_a).wait()
        pltpu.async_copy(rows_a, acc.at[cidx.at[n - 2]], ssa, add=True)
        pltpu.make_async_copy(y_hbm.at[ridx.at[n - 1]], rows_b,
                              sem_b).wait()
        pltpu.async_copy(rows_b, acc.at[cidx.at[n - 1]], ssb, add=True)
        pltpu.make_async_copy(rows_a, acc.at[cidx.at[n - 2]], ssa).wait()
        pltpu.make_async_copy(rows_b, acc.at[cidx.at[n - 1]], ssb).wait()

    for h in range(_CPW // _HCP):
        @pl.when(w < _LASTW)
        def _():
            base = w * _CPW + h * _HCP
            pltpu.sync_copy(edge3d_hbm.at[0, pl.ds(base, _HCP)], ridx)
            pltpu.sync_copy(edge3d_hbm.at[1, pl.ds(base, _HCP)], cidx)
            pipeline(_HCP)

        if h == 0:
            @pl.when(w == _LASTW)
            def _():
                base = _LASTW * _CPW
                pltpu.sync_copy(edge3d_hbm.at[0, pl.ds(base, _LASTN)],
                                ridx.at[pl.ds(0, _LASTN)])
                pltpu.sync_copy(edge3d_hbm.at[1, pl.ds(base, _LASTN)],
                                cidx.at[pl.ds(0, _LASTN)])
                pipeline(_LASTN)

    plsc.subcore_barrier()
    pltpu.sync_copy(acc.at[pl.ds(s * _RPS, _RPS)],
                    out_hbm.at[c, pl.ds(s * _RPS, _RPS)])


_R = 1000
_GRID = _N // _R


def _pre_a_body(nf, w0, b0, wc0, xo, h0o):
    x = lax.dot_general(nf[...], w0[...], (((1,), (1,)), ((), ())),
                        preferred_element_type=jnp.float32) + b0[...]
    xo[...] = x
    h0o[...] = lax.dot_general(x, wc0[...], (((1,), (1,)), ((), ())),
                               preferred_element_type=jnp.float32)


def _pre_b_body(degp, h0, y0o):
    d = degp[...]
    dinv = lax.rsqrt(d[0] + d[1] + 1.0)
    y0o[...] = dinv * h0[...]


def _mid_body(degp, x, y0, s0p, bc0, wc1, h0ro, y1o):
    d = degp[...]
    dinv = lax.rsqrt(d[0] + d[1] + 1.0)
    sp = s0p[...]
    t = dinv * (sp[0] + sp[1] + y0[...]) + bc0[...]
    h0r = jnp.maximum(t, 0.0)
    h0ro[...] = h0r
    emb = jnp.concatenate([x[...], h0r], axis=1)
    h1 = lax.dot_general(emb, wc1[...], (((1,), (1,)), ((), ())),
                         preferred_element_type=jnp.float32)
    y1o[...] = dinv * h1


def _fin_a_body(x, h0r, bt, pao, pacc):
    i = pl.program_id(0)
    emb = jnp.concatenate([x[...], h0r[...]], axis=1)
    seg = lax.broadcasted_iota(jnp.int32, (_R, _G), 1)
    onehot = jnp.where(bt[...] == seg, 1.0, 0.0).astype(jnp.float32)
    part = lax.dot_general(onehot, emb, (((0,), (0,)), ((), ())),
                           preferred_element_type=jnp.float32)

    @pl.when(i == 0)
    def _():
        pacc[...] = part

    @pl.when(i > 0)
    def _():
        pacc[...] = pacc[...] + part

    @pl.when(i == _GRID - 1)
    def _():
        pao[...] = pacc[...]


def _fin_b_body(degp, y1, s1p, bc1, bt, pa, wp1, bp1, wp2, bp2,
                out, pacc):
    i = pl.program_id(0)
    d = degp[...]
    dinv = lax.rsqrt(d[0] + d[1] + 1.0)
    sp = s1p[...]
    t = dinv * (sp[0] + sp[1] + y1[...]) + bc1[...]
    h1r = jnp.maximum(t, 0.0)
    seg = lax.broadcasted_iota(jnp.int32, (_R, _G), 1)
    onehot = jnp.where(bt[...] == seg, 1.0, 0.0).astype(jnp.float32)
    part = lax.dot_general(onehot, h1r, (((0,), (0,)), ((), ())),
                           preferred_element_type=jnp.float32)

    @pl.when(i == 0)
    def _():
        pacc[...] = part

    @pl.when(i > 0)
    def _():
        pacc[...] = pacc[...] + part

    @pl.when(i == _GRID - 1)
    def _():
        p = jnp.concatenate([pa[...], pacc[...]], axis=1)
        h = lax.dot_general(p, wp1[...], (((1,), (1,)), ((), ())),
                            preferred_element_type=jnp.float32) + bp1[...]
        h = jnp.where(h > 0, h, 0.1 * h)
        o = lax.dot_general(h, wp2[...], (((1,), (1,)), ((), ())),
                            preferred_element_type=jnp.float32) + bp2[...]
        m = jnp.max(o, axis=1, keepdims=True)
        lse = jnp.log(jnp.sum(jnp.exp(o - m), axis=1, keepdims=True)) + m
        out[...] = o - lse


def kernel(node_feature, edge_index, batch, W0, b0, Wc0, bc0, Wc1, bc1,
           Wp1, bp1, Wp2, bp2):
    f32 = jnp.float32
    edge3d = edge_index.reshape(2, _NCH, _CHUNK)
    zero1 = jnp.zeros((_RPS,), f32)
    zero2 = jnp.zeros((_RPS, _H), f32)
    bt2 = batch.reshape(_N, 1)

    dspec = pl.BlockSpec((_NCORE, _R, 1), lambda i: (0, i, 0))
    rspec = pl.BlockSpec((_R, _H), lambda i: (i, 0))
    sspec = pl.BlockSpec((_NCORE, _R, _H), lambda i: (0, i, 0))
    btspec = pl.BlockSpec((_R, 1), lambda i: (i, 0))

    deg_p = _deg_pass(edge3d, zero1)
    degp3 = deg_p.reshape(_NCORE, _NPAD, 1)

    x, h0 = pl.pallas_call(
        _pre_a_body,
        grid=(_GRID,),
        in_specs=[
            pl.BlockSpec((_R, _D), lambda i: (i, 0)),
            pl.BlockSpec((_H, _D), lambda i: (0, 0)),
            pl.BlockSpec((1, _H), lambda i: (0, 0)),
            pl.BlockSpec((_H, _H), lambda i: (0, 0)),
        ],
        out_specs=[rspec, rspec],
        out_shape=[jax.ShapeDtypeStruct((_N, _H), f32)] * 2,
    )(node_feature, W0, b0.reshape(1, _H), Wc0)

    y0 = pl.pallas_call(
        _pre_b_body,
        grid=(_GRID,),
        in_specs=[dspec, rspec],
        out_specs=rspec,
        out_shape=jax.ShapeDtypeStruct((_N, _H), f32),
    )(degp3, h0)

    s0_p = _edge_pass(edge3d, y0, zero2)

    h0r, y1 = pl.pallas_call(
        _mid_body,
        grid=(_GRID,),
        in_specs=[
            dspec, rspec, rspec, sspec,
            pl.BlockSpec((1, _H), lambda i: (0, 0)),
            pl.BlockSpec((_H, 2 * _H), lambda i: (0, 0)),
        ],
        out_specs=[rspec, rspec],
        out_shape=[jax.ShapeDtypeStruct((_N, _H), f32)] * 2,
    )(degp3, x, y0, s0_p, bc0.reshape(1, _H), Wc1)

    s1_p = _edge_pass(edge3d, y1, zero2)

    pooled_a = pl.pallas_call(
        _fin_a_body,
        grid=(_GRID,),
        in_specs=[rspec, rspec, btspec],
        out_specs=pl.BlockSpec((_G, 2 * _H), lambda i: (0, 0)),
        out_shape=jax.ShapeDtypeStruct((_G, 2 * _H), f32),
        scratch_shapes=[pltpu.VMEM((_G, 2 * _H), f32)],
    )(x, h0r, bt2)

    out = pl.pallas_call(
        _fin_b_body,
        grid=(_GRID,),
        in_specs=[
            dspec, rspec, sspec,
            pl.BlockSpec((1, _H), lambda i: (0, 0)),
            btspec,
            pl.BlockSpec((_G, 2 * _H), lambda i: (0, 0)),
            pl.BlockSpec((_H, 3 * _H), lambda i: (0, 0)),
            pl.BlockSpec((1, _H), lambda i: (0, 0)),
            pl.BlockSpec((_OUT, _H), lambda i: (0, 0)),
            pl.BlockSpec((1, _OUT), lambda i: (0, 0)),
        ],
        out_specs=pl.BlockSpec((_G, _OUT), lambda i: (0, 0)),
        out_shape=jax.ShapeDtypeStruct((_G, _OUT), f32),
        scratch_shapes=[pltpu.VMEM((_G, _H), f32)],
    )(degp3, y1, s1_p, bc1.reshape(1, _H), bt2, pooled_a,
      Wp1, bp1.reshape(1, _H), Wp2, bp2.reshape(1, _OUT))
    return out


# speedup vs baseline: 1.2305x; 1.2305x over previous
import functools

import jax
import jax.numpy as jnp
from jax import lax
from jax.experimental import pallas as pl
from jax.experimental.pallas import tpu as pltpu
from jax.experimental.pallas import tpu_sc as plsc

_N = 10000
_E = 320000
_D = 128
_H = 128
_OUT = 32
_G = 64

_NCORE = 2
_NSUB = 16
_NW = _NCORE * _NSUB
_NPAD = 10240
_RPS = _NPAD // _NSUB
_CHUNK = 128
_NCH = _E // _CHUNK
_CPW = 80
_HCP = 40
_LASTW = _NCH // _CPW
_LASTN = _NCH - _LASTW * _CPW

_mesh = plsc.VectorSubcoreMesh(core_axis_name="c", subcore_axis_name="s")


@functools.partial(
    pl.kernel,
    out_type=jax.ShapeDtypeStruct((_NCORE, _NPAD), jnp.float32),
    mesh=_mesh,
    scratch_types=[
        pltpu.VMEM((_CPW, _CHUNK), jnp.int32),
        pltpu.VMEM((_CHUNK,), jnp.float32),
        pltpu.VMEM_SHARED((_NPAD,), jnp.float32),
        pltpu.SemaphoreType.DMA,
        pltpu.SemaphoreType.DMA,
    ],
)
def _deg_pass(edge3d_hbm, zero1_hbm, out_hbm, cidx, ones_v, acc, dsa, dsb):
    c = lax.axis_index("c")
    s = lax.axis_index("s")
    w = s * _NCORE + c
    for i in range(_CHUNK // 16):
        ones_v[pl.ds(i * 16, 16)] = jnp.ones((16,), jnp.float32)
    pltpu.sync_copy(zero1_hbm, acc.at[pl.ds(s * _RPS, _RPS)])

    @pl.when(w < _LASTW)
    def _():
        pltpu.sync_copy(edge3d_hbm.at[1, pl.ds(w * _CPW, _CPW)], cidx)

    @pl.when(w == _LASTW)
    def _():
        pltpu.sync_copy(edge3d_hbm.at[1, pl.ds(_LASTW * _CPW, _LASTN)],
                        cidx.at[pl.ds(0, _LASTN)])

    plsc.subcore_barrier()

    def dpipe(n):
        pltpu.async_copy(ones_v, acc.at[cidx.at[0]], dsa, add=True)
        pltpu.async_copy(ones_v, acc.at[cidx.at[1]], dsb, add=True)

        def body(k2, carry):
            kk = k2 * 2
            pltpu.make_async_copy(ones_v, acc.at[cidx.at[kk - 2]],
                                  dsa).wait()
            pltpu.async_copy(ones_v, acc.at[cidx.at[kk]], dsa, add=True)
            pltpu.make_async_copy(ones_v, acc.at[cidx.at[kk - 1]],
                                  dsb).wait()
            pltpu.async_copy(ones_v, acc.at[cidx.at[kk + 1]], dsb,
                             add=True)
            return carry

        lax.fori_loop(1, n // 2, body, 0)
        pltpu.make_async_copy(ones_v, acc.at[cidx.at[n - 2]], dsa).wait()
        pltpu.make_async_copy(ones_v, acc.at[cidx.at[n - 1]], dsb).wait()

    @pl.when(w < _LASTW)
    def _():
        dpipe(_CPW)

    @pl.when(w == _LASTW)
    def _():
        dpipe(_LASTN)

    plsc.subcore_barrier()
    pltpu.sync_copy(acc.at[pl.ds(s * _RPS, _RPS)],
                    out_hbm.at[c, pl.ds(s * _RPS, _RPS)])


@functools.partial(
    pl.kernel,
    out_type=jax.ShapeDtypeStruct((_NCORE, _NPAD, _H), jnp.float32),
    mesh=_mesh,
    scratch_types=[
        pltpu.VMEM((_HCP, _CHUNK), jnp.int32),
        pltpu.VMEM((_HCP, _CHUNK), jnp.int32),
        pltpu.VMEM((_CHUNK, _H), jnp.float32),
        pltpu.VMEM((_CHUNK, _H), jnp.float32),
        pltpu.VMEM_SHARED((_NPAD, _H), jnp.float32),
        pltpu.SemaphoreType.DMA,
        pltpu.SemaphoreType.DMA,
    ],
)
def _edge_pass(edge3d_hbm, y_hbm, zero2_hbm, out_hbm,
               ridx, cidx, rows_a, rows_b, acc, sem_a, sem_b):
    c = lax.axis_index("c")
    s = lax.axis_index("s")
    w = s * _NCORE + c
    pltpu.sync_copy(zero2_hbm, acc.at[pl.ds(s * _RPS, _RPS)])
    plsc.subcore_barrier()

    def pipeline(n):
        pltpu.async_copy(y_hbm.at[ridx.at[0]], rows_a, sem_a)

        def body(k2, carry):
            kk = k2 * 2
            pltpu.async_copy(y_hbm.at[ridx.at[kk + 1]], rows_b, sem_b)
            pltpu.make_async_copy(y_hbm.at[ridx.at[kk]], rows_a,
                                  sem_a).wait()
            pltpu.sync_copy(rows_a, acc.at[cidx.at[kk]], add=True)
            pltpu.async_copy(y_hbm.at[ridx.at[kk + 2]], rows_a, sem_a)
            pltpu.make_async_copy(y_hbm.at[ridx.at[kk + 1]], rows_b,
                                  sem_b).wait()
            pltpu.sync_copy(rows_b, acc.at[cidx.at[kk + 1]], add=True)
            return carry

        lax.fori_loop(0, n // 2 - 1, body, 0)
        pltpu.async_copy(y_hbm.at[ridx.at[n - 1]], rows_b, sem_b)
        pltpu.make_async_copy(y_hbm.at[ridx.at[n - 2]], rows_a,
                              sem_a).wait()
        pltpu.sync_copy(rows_a, acc.at[cidx.at[n - 2]], add=True)
        pltpu.make_async_copy(y_hbm.at[ridx.at[n - 1]], rows_b,
                              sem_b).wait()
        pltpu.sync_copy(rows_b, acc.at[cidx.at[n - 1]], add=True)

    for h in range(_CPW // _HCP):
        @pl.when(w < _LASTW)
        def _():
            base = w * _CPW + h * _HCP
            pltpu.sync_copy(edge3d_hbm.at[0, pl.ds(base, _HCP)], ridx)
            pltpu.sync_copy(edge3d_hbm.at[1, pl.ds(base, _HCP)], cidx)
            pipeline(_HCP)

        if h == 0:
            @pl.when(w == _LASTW)
            def _():
                base = _LASTW * _CPW
                pltpu.sync_copy(edge3d_hbm.at[0, pl.ds(base, _LASTN)],
                                ridx.at[pl.ds(0, _LASTN)])
                pltpu.sync_copy(edge3d_hbm.at[1, pl.ds(base, _LASTN)],
                                cidx.at[pl.ds(0, _LASTN)])
                pipeline(_LASTN)

    plsc.subcore_barrier()
    pltpu.sync_copy(acc.at[pl.ds(s * _RPS, _RPS)],
                    out_hbm.at[c, pl.ds(s * _RPS, _RPS)])


_R = 1000
_GRID = _N // _R


def _pre_a_body(nf, w0, b0, wc0, xo, h0o):
    x = lax.dot_general(nf[...], w0[...], (((1,), (1,)), ((), ())),
                        preferred_element_type=jnp.float32) + b0[...]
    xo[...] = x
    h0o[...] = lax.dot_general(x, wc0[...], (((1,), (1,)), ((), ())),
                               preferred_element_type=jnp.float32)


def _pre_b_body(degp, h0, y0o):
    d = degp[...]
    dinv = lax.rsqrt(d[0] + d[1] + 1.0)
    y0o[...] = dinv * h0[...]


def _mid_a_body(x, wc1a, t1o):
    t1o[...] = lax.dot_general(x[...], wc1a[...], (((1,), (1,)), ((), ())),
                               preferred_element_type=jnp.float32)


def _mid_b_body(degp, t1, y0, s0p, bc0, wc1b, h0ro, y1o):
    d = degp[...]
    dinv = lax.rsqrt(d[0] + d[1] + 1.0)
    sp = s0p[...]
    t = dinv * (sp[0] + sp[1] + y0[...]) + bc0[...]
    h0r = jnp.maximum(t, 0.0)
    h0ro[...] = h0r
    h1 = t1[...] + lax.dot_general(h0r, wc1b[...], (((1,), (1,)), ((), ())),
                                   preferred_element_type=jnp.float32)
    y1o[...] = dinv * h1


def _fin_a_body(x, h0r, bt, pao, pacc):
    i = pl.program_id(0)
    emb = jnp.concatenate([x[...], h0r[...]], axis=1)
    seg = lax.broadcasted_iota(jnp.int32, (_R, _G), 1)
    onehot = jnp.where(bt[...] == seg, 1.0, 0.0).astype(jnp.float32)
    part = lax.dot_general(onehot, emb, (((0,), (0,)), ((), ())),
                           preferred_element_type=jnp.float32)

    @pl.when(i == 0)
    def _():
        pacc[...] = part

    @pl.when(i > 0)
    def _():
        pacc[...] = pacc[...] + part

    @pl.when(i == _GRID - 1)
    def _():
        pao[...] = pacc[...]


def _fin_b_body(degp, y1, s1p, bc1, bt, pa, wp1, bp1, wp2, bp2,
                out, pacc):
    i = pl.program_id(0)
    d = degp[...]
    dinv = lax.rsqrt(d[0] + d[1] + 1.0)
    sp = s1p[...]
    t = dinv * (sp[0] + sp[1] + y1[...]) + bc1[...]
    h1r = jnp.maximum(t, 0.0)
    seg = lax.broadcasted_iota(jnp.int32, (_R, _G), 1)
    onehot = jnp.where(bt[...] == seg, 1.0, 0.0).astype(jnp.float32)
    part = lax.dot_general(onehot, h1r, (((0,), (0,)), ((), ())),
                           preferred_element_type=jnp.float32)

    @pl.when(i == 0)
    def _():
        pacc[...] = part

    @pl.when(i > 0)
    def _():
        pacc[...] = pacc[...] + part

    @pl.when(i == _GRID - 1)
    def _():
        p = jnp.concatenate([pa[...], pacc[...]], axis=1)
        h = lax.dot_general(p, wp1[...], (((1,), (1,)), ((), ())),
                            preferred_element_type=jnp.float32) + bp1[...]
        h = jnp.where(h > 0, h, 0.1 * h)
        o = lax.dot_general(h, wp2[...], (((1,), (1,)), ((), ())),
                            preferred_element_type=jnp.float32) + bp2[...]
        m = jnp.max(o, axis=1, keepdims=True)
        lse = jnp.log(jnp.sum(jnp.exp(o - m), axis=1, keepdims=True)) + m
        out[...] = o - lse


def kernel(node_feature, edge_index, batch, W0, b0, Wc0, bc0, Wc1, bc1,
           Wp1, bp1, Wp2, bp2):
    f32 = jnp.float32
    edge3d = edge_index.reshape(2, _NCH, _CHUNK)
    zero1 = jnp.zeros((_RPS,), f32)
    zero2 = jnp.zeros((_RPS, _H), f32)
    bt2 = batch.reshape(_N, 1)

    dspec = pl.BlockSpec((_NCORE, _R, 1), lambda i: (0, i, 0))
    rspec = pl.BlockSpec((_R, _H), lambda i: (i, 0))
    sspec = pl.BlockSpec((_NCORE, _R, _H), lambda i: (0, i, 0))
    btspec = pl.BlockSpec((_R, 1), lambda i: (i, 0))

    deg_p = _deg_pass(edge3d, zero1)
    degp3 = deg_p.reshape(_NCORE, _NPAD, 1)

    x, h0 = pl.pallas_call(
        _pre_a_body,
        grid=(_GRID,),
        in_specs=[
            pl.BlockSpec((_R, _D), lambda i: (i, 0)),
            pl.BlockSpec((_H, _D), lambda i: (0, 0)),
            pl.BlockSpec((1, _H), lambda i: (0, 0)),
            pl.BlockSpec((_H, _H), lambda i: (0, 0)),
        ],
        out_specs=[rspec, rspec],
        out_shape=[jax.ShapeDtypeStruct((_N, _H), f32)] * 2,
    )(node_feature, W0, b0.reshape(1, _H), Wc0)

    y0 = pl.pallas_call(
        _pre_b_body,
        grid=(_GRID,),
        in_specs=[dspec, rspec],
        out_specs=rspec,
        out_shape=jax.ShapeDtypeStruct((_N, _H), f32),
    )(degp3, h0)

    s0_p = _edge_pass(edge3d, y0, zero2)

    t1 = pl.pallas_call(
        _mid_a_body,
        grid=(_GRID,),
        in_specs=[rspec, pl.BlockSpec((_H, _H), lambda i: (0, 0))],
        out_specs=rspec,
        out_shape=jax.ShapeDtypeStruct((_N, _H), f32),
    )(x, Wc1[:, :_H])

    h0r, y1 = pl.pallas_call(
        _mid_b_body,
        grid=(_GRID,),
        in_specs=[
            dspec, rspec, rspec, sspec,
            pl.BlockSpec((1, _H), lambda i: (0, 0)),
            pl.BlockSpec((_H, _H), lambda i: (0, 0)),
        ],
        out_specs=[rspec, rspec],
        out_shape=[jax.ShapeDtypeStruct((_N, _H), f32)] * 2,
    )(degp3, t1, y0, s0_p, bc0.reshape(1, _H), Wc1[:, _H:])

    s1_p = _edge_pass(edge3d, y1, zero2)

    pooled_a = pl.pallas_call(
        _fin_a_body,
        grid=(_GRID,),
        in_specs=[rspec, rspec, btspec],
        out_specs=pl.BlockSpec((_G, 2 * _H), lambda i: (0, 0)),
        out_shape=jax.ShapeDtypeStruct((_G, 2 * _H), f32),
        scratch_shapes=[pltpu.VMEM((_G, 2 * _H), f32)],
    )(x, h0r, bt2)

    out = pl.pallas_call(
        _fin_b_body,
        grid=(_GRID,),
        in_specs=[
            dspec, rspec, sspec,
            pl.BlockSpec((1, _H), lambda i: (0, 0)),
            btspec,
            pl.BlockSpec((_G, 2 * _H), lambda i: (0, 0)),
            pl.BlockSpec((_H, 3 * _H), lambda i: (0, 0)),
            pl.BlockSpec((1, _H), lambda i: (0, 0)),
            pl.BlockSpec((_OUT, _H), lambda i: (0, 0)),
            pl.BlockSpec((1, _OUT), lambda i: (0, 0)),
        ],
        out_specs=pl.BlockSpec((_G, _OUT), lambda i: (0, 0)),
        out_shape=jax.ShapeDtypeStruct((_G, _OUT), f32),
        scratch_shapes=[pltpu.VMEM((_G, _H), f32)],
    )(degp3, y1, s1_p, bc1.reshape(1, _H), bt2, pooled_a,
      Wp1, bp1.reshape(1, _H), Wp2, bp2.reshape(1, _OUT))
    return out


# speedup vs baseline: 1.2819x; 1.0418x over previous
import functools

import jax
import jax.numpy as jnp
from jax import lax
from jax.experimental import pallas as pl
from jax.experimental.pallas import tpu as pltpu
from jax.experimental.pallas import tpu_sc as plsc

_N = 10000
_E = 320000
_D = 128
_H = 128
_OUT = 32
_G = 64

_NCORE = 2
_NSUB = 16
_NW = _NCORE * _NSUB
_NPAD = 10240
_RPS = _NPAD // _NSUB
_CHUNK = 128
_NCH = _E // _CHUNK
_CPW = 80
_HCP = 40
_LASTW = _NCH // _CPW
_LASTN = _NCH - _LASTW * _CPW

_mesh = plsc.VectorSubcoreMesh(core_axis_name="c", subcore_axis_name="s")


@functools.partial(
    pl.kernel,
    out_type=jax.ShapeDtypeStruct((_NCORE, _NPAD), jnp.float32),
    mesh=_mesh,
    scratch_types=[
        pltpu.VMEM((_CPW, _CHUNK), jnp.int32),
        pltpu.VMEM((_CHUNK,), jnp.float32),
        pltpu.VMEM_SHARED((_NPAD,), jnp.float32),
        pltpu.SemaphoreType.DMA,
        pltpu.SemaphoreType.DMA,
    ],
)
def _deg_pass(edge3d_hbm, zero1_hbm, out_hbm, cidx, ones_v, acc, dsa, dsb):
    c = lax.axis_index("c")
    s = lax.axis_index("s")
    w = s * _NCORE + c
    for i in range(_CHUNK // 16):
        ones_v[pl.ds(i * 16, 16)] = jnp.ones((16,), jnp.float32)
    pltpu.sync_copy(zero1_hbm, acc.at[pl.ds(s * _RPS, _RPS)])

    @pl.when(w < _LASTW)
    def _():
        pltpu.sync_copy(edge3d_hbm.at[1, pl.ds(w * _CPW, _CPW)], cidx)

    @pl.when(w == _LASTW)
    def _():
        pltpu.sync_copy(edge3d_hbm.at[1, pl.ds(_LASTW * _CPW, _LASTN)],
                        cidx.at[pl.ds(0, _LASTN)])

    plsc.subcore_barrier()

    def dpipe(n):
        pltpu.async_copy(ones_v, acc.at[cidx.at[0]], dsa, add=True)
        pltpu.async_copy(ones_v, acc.at[cidx.at[1]], dsb, add=True)

        def body(k2, carry):
            kk = k2 * 2
            pltpu.make_async_copy(ones_v, acc.at[cidx.at[kk - 2]],
                                  dsa).wait()
            pltpu.async_copy(ones_v, acc.at[cidx.at[kk]], dsa, add=True)
            pltpu.make_async_copy(ones_v, acc.at[cidx.at[kk - 1]],
                                  dsb).wait()
            pltpu.async_copy(ones_v, acc.at[cidx.at[kk + 1]], dsb,
                             add=True)
            return carry

        lax.fori_loop(1, n // 2, body, 0)
        pltpu.make_async_copy(ones_v, acc.at[cidx.at[n - 2]], dsa).wait()
        pltpu.make_async_copy(ones_v, acc.at[cidx.at[n - 1]], dsb).wait()

    @pl.when(w < _LASTW)
    def _():
        dpipe(_CPW)

    @pl.when(w == _LASTW)
    def _():
        dpipe(_LASTN)

    plsc.subcore_barrier()
    pltpu.sync_copy(acc.at[pl.ds(s * _RPS, _RPS)],
                    out_hbm.at[c, pl.ds(s * _RPS, _RPS)])


@functools.partial(
    pl.kernel,
    out_type=jax.ShapeDtypeStruct((_NCORE, _NPAD, _H), jnp.float32),
    mesh=_mesh,
    scratch_types=[
        pltpu.VMEM((_HCP, _CHUNK), jnp.int32),
        pltpu.VMEM((_HCP, _CHUNK), jnp.int32),
        pltpu.VMEM((_CHUNK, _H), jnp.float32),
        pltpu.VMEM((_CHUNK, _H), jnp.float32),
        pltpu.VMEM_SHARED((_NPAD, _H), jnp.float32),
        pltpu.SemaphoreType.DMA,
        pltpu.SemaphoreType.DMA,
    ],
)
def _edge_pass(edge3d_hbm, y_hbm, out_hbm,
               ridx, cidx, rows_a, rows_b, acc, sem_a, sem_b):
    c = lax.axis_index("c")
    s = lax.axis_index("s")
    w = s * _NCORE + c

    def zbody(i, carry):
        for j in range(_H // 16):
            rows_a[i, pl.ds(j * 16, 16)] = jnp.zeros((16,), jnp.float32)
        return carry

    lax.fori_loop(0, _CHUNK, zbody, 0)
    for r in range(_RPS // _CHUNK):
        pltpu.sync_copy(rows_a,
                        acc.at[pl.ds(s * _RPS + r * _CHUNK, _CHUNK)])
    plsc.subcore_barrier()

    def pipeline(n):
        pltpu.async_copy(y_hbm.at[ridx.at[0]], rows_a, sem_a)

        def body(k2, carry):
            kk = k2 * 2
            pltpu.async_copy(y_hbm.at[ridx.at[kk + 1]], rows_b, sem_b)
            pltpu.make_async_copy(y_hbm.at[ridx.at[kk]], rows_a,
                                  sem_a).wait()
            pltpu.sync_copy(rows_a, acc.at[cidx.at[kk]], add=True)
            pltpu.async_copy(y_hbm.at[ridx.at[kk + 2]], rows_a, sem_a)
            pltpu.make_async_copy(y_hbm.at[ridx.at[kk + 1]], rows_b,
                                  sem_b).wait()
            pltpu.sync_copy(rows_b, acc.at[cidx.at[kk + 1]], add=True)
            return carry

        lax.fori_loop(0, n // 2 - 1, body, 0)
        pltpu.async_copy(y_hbm.at[ridx.at[n - 1]], rows_b, sem_b)
        pltpu.make_async_copy(y_hbm.at[ridx.at[n - 2]], rows_a,
                              sem_a).wait()
        pltpu.sync_copy(rows_a, acc.at[cidx.at[n - 2]], add=True)
        pltpu.make_async_copy(y_hbm.at[ridx.at[n - 1]], rows_b,
                              sem_b).wait()
        pltpu.sync_copy(rows_b, acc.at[cidx.at[n - 1]], add=True)

    for h in range(_CPW // _HCP):
        @pl.when(w < _LASTW)
        def _():
            base = w * _CPW + h * _HCP
            pltpu.sync_copy(edge3d_hbm.at[0, pl.ds(base, _HCP)], ridx)
            pltpu.sync_copy(edge3d_hbm.at[1, pl.ds(base, _HCP)], cidx)
            pipeline(_HCP)

        if h == 0:
            @pl.when(w == _LASTW)
            def _():
                base = _LASTW * _CPW
                pltpu.sync_copy(edge3d_hbm.at[0, pl.ds(base, _LASTN)],
                                ridx.at[pl.ds(0, _LASTN)])
                pltpu.sync_copy(edge3d_hbm.at[1, pl.ds(base, _LASTN)],
                                cidx.at[pl.ds(0, _LASTN)])
                pipeline(_LASTN)

    plsc.subcore_barrier()
    pltpu.sync_copy(acc.at[pl.ds(s * _RPS, _RPS)],
                    out_hbm.at[c, pl.ds(s * _RPS, _RPS)])


_R = 1000
_GRID = _N // _R


def _pre_a_body(nf, w0, b0, wc0, xo, h0o):
    x = lax.dot_general(nf[...], w0[...], (((1,), (1,)), ((), ())),
                        preferred_element_type=jnp.float32) + b0[...]
    xo[...] = x
    h0o[...] = lax.dot_general(x, wc0[...], (((1,), (1,)), ((), ())),
                               preferred_element_type=jnp.float32)


def _pre_b_body(degp, h0, y0o):
    d = degp[...]
    dinv = lax.rsqrt(d[0] + d[1] + 1.0)
    y0o[...] = dinv * h0[...]


def _mid_a_body(x, wc1a, t1o):
    t1o[...] = lax.dot_general(x[...], wc1a[...], (((1,), (1,)), ((), ())),
                               preferred_element_type=jnp.float32)


def _mid_b_body(degp, t1, y0, s0p, bc0, wc1b, h0ro, y1o):
    d = degp[...]
    dinv = lax.rsqrt(d[0] + d[1] + 1.0)
    sp = s0p[...]
    t = dinv * (sp[0] + sp[1] + y0[...]) + bc0[...]
    h0r = jnp.maximum(t, 0.0)
    h0ro[...] = h0r
    h1 = t1[...] + lax.dot_general(h0r, wc1b[...], (((1,), (1,)), ((), ())),
                                   preferred_element_type=jnp.float32)
    y1o[...] = dinv * h1


def _fin_a_body(x, h0r, bt, pao, pacc):
    i = pl.program_id(0)
    emb = jnp.concatenate([x[...], h0r[...]], axis=1)
    seg = lax.broadcasted_iota(jnp.int32, (_R, _G), 1)
    onehot = jnp.where(bt[...] == seg, 1.0, 0.0).astype(jnp.float32)
    part = lax.dot_general(onehot, emb, (((0,), (0,)), ((), ())),
                           preferred_element_type=jnp.float32)

    @pl.when(i == 0)
    def _():
        pacc[...] = part

    @pl.when(i > 0)
    def _():
        pacc[...] = pacc[...] + part

    @pl.when(i == _GRID - 1)
    def _():
        pao[...] = pacc[...]


def _fin_b_body(degp, y1, s1p, bc1, bt, pa, wp1, bp1, wp2, bp2,
                out, pacc):
    i = pl.program_id(0)
    d = degp[...]
    dinv = lax.rsqrt(d[0] + d[1] + 1.0)
    sp = s1p[...]
    t = dinv * (sp[0] + sp[1] + y1[...]) + bc1[...]
    h1r = jnp.maximum(t, 0.0)
    seg = lax.broadcasted_iota(jnp.int32, (_R, _G), 1)
    onehot = jnp.where(bt[...] == seg, 1.0, 0.0).astype(jnp.float32)
    part = lax.dot_general(onehot, h1r, (((0,), (0,)), ((), ())),
                           preferred_element_type=jnp.float32)

    @pl.when(i == 0)
    def _():
        pacc[...] = part

    @pl.when(i > 0)
    def _():
        pacc[...] = pacc[...] + part

    @pl.when(i == _GRID - 1)
    def _():
        p = jnp.concatenate([pa[...], pacc[...]], axis=1)
        h = lax.dot_general(p, wp1[...], (((1,), (1,)), ((), ())),
                            preferred_element_type=jnp.float32) + bp1[...]
        h = jnp.where(h > 0, h, 0.1 * h)
        o = lax.dot_general(h, wp2[...], (((1,), (1,)), ((), ())),
                            preferred_element_type=jnp.float32) + bp2[...]
        m = jnp.max(o, axis=1, keepdims=True)
        lse = jnp.log(jnp.sum(jnp.exp(o - m), axis=1, keepdims=True)) + m
        out[...] = o - lse


def kernel(node_feature, edge_index, batch, W0, b0, Wc0, bc0, Wc1, bc1,
           Wp1, bp1, Wp2, bp2):
    f32 = jnp.float32
    edge3d = edge_index.reshape(2, _NCH, _CHUNK)
    zero1 = jnp.zeros((_RPS,), f32)
    bt2 = batch.reshape(_N, 1)

    dspec = pl.BlockSpec((_NCORE, _R, 1), lambda i: (0, i, 0))
    rspec = pl.BlockSpec((_R, _H), lambda i: (i, 0))
    sspec = pl.BlockSpec((_NCORE, _R, _H), lambda i: (0, i, 0))
    btspec = pl.BlockSpec((_R, 1), lambda i: (i, 0))

    deg_p = _deg_pass(edge3d, zero1)
    degp3 = deg_p.reshape(_NCORE, _NPAD, 1)

    x, h0 = pl.pallas_call(
        _pre_a_body,
        grid=(_GRID,),
        in_specs=[
            pl.BlockSpec((_R, _D), lambda i: (i, 0)),
            pl.BlockSpec((_H, _D), lambda i: (0, 0)),
            pl.BlockSpec((1, _H), lambda i: (0, 0)),
            pl.BlockSpec((_H, _H), lambda i: (0, 0)),
        ],
        out_specs=[rspec, rspec],
        out_shape=[jax.ShapeDtypeStruct((_N, _H), f32)] * 2,
    )(node_feature, W0, b0.reshape(1, _H), Wc0)

    y0 = pl.pallas_call(
        _pre_b_body,
        grid=(_GRID,),
        in_specs=[dspec, rspec],
        out_specs=rspec,
        out_shape=jax.ShapeDtypeStruct((_N, _H), f32),
    )(degp3, h0)

    s0_p = _edge_pass(edge3d, y0)

    t1 = pl.pallas_call(
        _mid_a_body,
        grid=(_GRID,),
        in_specs=[rspec, pl.BlockSpec((_H, _H), lambda i: (0, 0))],
        out_specs=rspec,
        out_shape=jax.ShapeDtypeStruct((_N, _H), f32),
    )(x, Wc1[:, :_H])

    h0r, y1 = pl.pallas_call(
        _mid_b_body,
        grid=(_GRID,),
        in_specs=[
            dspec, rspec, rspec, sspec,
            pl.BlockSpec((1, _H), lambda i: (0, 0)),
            pl.BlockSpec((_H, _H), lambda i: (0, 0)),
        ],
        out_specs=[rspec, rspec],
        out_shape=[jax.ShapeDtypeStruct((_N, _H), f32)] * 2,
    )(degp3, t1, y0, s0_p, bc0.reshape(1, _H), Wc1[:, _H:])

    s1_p = _edge_pass(edge3d, y1)

    pooled_a = pl.pallas_call(
        _fin_a_body,
        grid=(_GRID,),
        in_specs=[rspec, rspec, btspec],
        out_specs=pl.BlockSpec((_G, 2 * _H), lambda i: (0, 0)),
        out_shape=jax.ShapeDtypeStruct((_G, 2 * _H), f32),
        scratch_shapes=[pltpu.VMEM((_G, 2 * _H), f32)],
    )(x, h0r, bt2)

    out = pl.pallas_call(
        _fin_b_body,
        grid=(_GRID,),
        in_specs=[
            dspec, rspec, sspec,
            pl.BlockSpec((1, _H), lambda i: (0, 0)),
            btspec,
            pl.BlockSpec((_G, 2 * _H), lambda i: (0, 0)),
            pl.BlockSpec((_H, 3 * _H), lambda i: (0, 0)),
            pl.BlockSpec((1, _H), lambda i: (0, 0)),
            pl.BlockSpec((_OUT, _H), lambda i: (0, 0)),
            pl.BlockSpec((1, _OUT), lambda i: (0, 0)),
        ],
        out_specs=pl.BlockSpec((_G, _OUT), lambda i: (0, 0)),
        out_shape=jax.ShapeDtypeStruct((_G, _OUT), f32),
        scratch_shapes=[pltpu.VMEM((_G, _H), f32)],
    )(degp3, y1, s1_p, bc1.reshape(1, _H), bt2, pooled_a,
      Wp1, bp1.reshape(1, _H), Wp2, bp2.reshape(1, _OUT))
    return out
